# Initial kernel scaffold; baseline (speedup 1.0000x reference)
import jax
import jax.numpy as jnp
from jax.experimental import pallas as pl


def kernel(features, coords, batch_indices):
    raise NotImplementedError("write your pallas kernel here")


# speedup vs baseline: 2.9679x; 2.9679x over previous
import jax
import jax.numpy as jnp
from jax.experimental import pallas as pl
from jax.experimental.pallas import tpu as pltpu

N = 1000000
B = 8
D = 64
NS = 16
NSEG = B * NS

CHUNK = 4000
NCHUNK = N // CHUNK


def _stats_kernel(bic_ref, coords_ref, out_ref, acc_ref):
    i = pl.program_id(0)

    @pl.when(i == 0)
    def _():
        acc_ref[...] = jnp.zeros_like(acc_ref)

    bic = bic_ref[...]
    ci = coords_ref[...]
    for b in range(B):
        m = bic == b
        acc_ref[b:b + 1, 0:3] += jnp.sum(jnp.where(m, ci, 0), axis=0,
                                         keepdims=True)
        acc_ref[b:b + 1, 3:4] += jnp.sum(m.astype(jnp.int32), axis=0,
                                         keepdims=True)

    @pl.when(i == NCHUNK - 1)
    def _():
        out_ref[...] = acc_ref[...]


def _main_kernel(stats_ref, bic_ref, coords_ref, feat_ref, out_ref,
                 facc_ref, cacc_ref):
    i = pl.program_id(0)

    @pl.when(i == 0)
    def _():
        facc_ref[...] = jnp.zeros_like(facc_ref)
        cacc_ref[...] = jnp.zeros_like(cacc_ref)

    stats = stats_ref[...].astype(jnp.float32)
    mean = stats[:, 0:3] / jnp.maximum(stats[:, 3:4], 1.0)
    bic = bic_ref[...]
    cf = coords_ref[...].astype(jnp.float32)
    mg = jnp.broadcast_to(mean[0:1, :], (CHUNK, 3))
    for b in range(1, B):
        mg = jnp.where(bic == b, mean[b:b + 1, :], mg)
    c = cf - mg
    d2 = jnp.sum(c * c, axis=1, keepdims=True)
    bev = jnp.sqrt(d2) * 0.05
    kk = jax.lax.broadcasted_iota(jnp.int32, (1, NS), 1).astype(jnp.float32) + 1.0
    rk = 4.0 + 3.25 * kk
    sidx = jnp.sum((bev >= rk).astype(jnp.int32), axis=1, keepdims=True)
    seg = jnp.where(sidx < NS, bic * NS + sidx, NSEG)
    ohs = (jax.lax.broadcasted_iota(jnp.int32, (CHUNK, NSEG), 1) == seg
           ).astype(jnp.float32)
    feat = feat_ref[...]
    facc_ref[...] += jax.lax.dot_general(
        ohs, feat, (((0,), (0,)), ((), ())), preferred_element_type=jnp.float32)
    cacc_ref[...] += jax.lax.dot_general(
        ohs, jnp.ones((CHUNK, 8), jnp.float32), (((0,), (0,)), ((), ())),
        preferred_element_type=jnp.float32)

    @pl.when(i == NCHUNK - 1)
    def _():
        cnt = cacc_ref[:, 0:1]
        out_ref[...] = jnp.where(cnt > 0, facc_ref[...] / jnp.maximum(cnt, 1.0),
                                 0.0)


def kernel(features, coords, batch_indices):
    bi_col = batch_indices.astype(jnp.int32).reshape(N, 1)
    stats = pl.pallas_call(
        _stats_kernel,
        grid=(NCHUNK,),
        in_specs=[
            pl.BlockSpec((CHUNK, 1), lambda i: (i, 0)),
            pl.BlockSpec((CHUNK, 3), lambda i: (i, 0)),
        ],
        out_specs=pl.BlockSpec((B, 4), lambda i: (0, 0)),
        out_shape=jax.ShapeDtypeStruct((B, 4), jnp.int32),
        scratch_shapes=[pltpu.VMEM((B, 4), jnp.int32)],
    )(bi_col, coords)
    out = pl.pallas_call(
        _main_kernel,
        grid=(NCHUNK,),
        in_specs=[
            pl.BlockSpec((B, 4), lambda i: (0, 0)),
            pl.BlockSpec((CHUNK, 1), lambda i: (i, 0)),
            pl.BlockSpec((CHUNK, 3), lambda i: (i, 0)),
            pl.BlockSpec((CHUNK, D), lambda i: (i, 0)),
        ],
        out_specs=pl.BlockSpec((NSEG, D), lambda i: (0, 0)),
        out_shape=jax.ShapeDtypeStruct((NSEG, D), jnp.float32),
        scratch_shapes=[
            pltpu.VMEM((NSEG, D), jnp.float32),
            pltpu.VMEM((NSEG, 8), jnp.float32),
        ],
    )(stats, bi_col, coords, features)
    return out.reshape(B, NS * D)


# speedup vs baseline: 9.0605x; 3.0529x over previous
import functools

import jax
import jax.numpy as jnp
import numpy as np
from jax import lax
from jax.experimental import pallas as pl
from jax.experimental.pallas import tpu as pltpu
from jax.experimental.pallas import tpu_sc as plsc

N = 1000000
B = 8
D = 64
NS = 16
NSEG = B * NS

NW = 32
BLK = 2000
NBLK = N // BLK
VPB = BLK // 16
NBLK_BASE = NBLK // NW
NBLK_EXTRA = NBLK - NBLK_BASE * NW

CHUNK = 4000
NCHUNK = N // CHUNK


def _exact_d2_thresholds():
    v05 = np.float32(0.05)
    thr = []
    for k in range(1, 17):
        rk = np.float32(4.0 + 3.25 * k)

        def pred(v):
            return np.float32(np.sqrt(np.float32(v)) * v05) >= rk

        v = np.float32((80.0 + 65.0 * k) ** 2)
        if pred(v):
            while pred(np.nextafter(v, np.float32(0.0))):
                v = np.nextafter(v, np.float32(0.0))
        else:
            while not pred(v):
                v = np.nextafter(v, np.float32(np.inf))
        thr.append(float(v))
    return thr


_THR = _exact_d2_thresholds()

_MESH = plsc.VectorSubcoreMesh(core_axis_name="c", subcore_axis_name="s")


def _wid():
    return lax.axis_index("s") * 2 + lax.axis_index("c")


def _nblk(wid):
    return NBLK_BASE + jnp.where(wid < NBLK_EXTRA, 1, 0)


@functools.partial(
    pl.kernel,
    out_type=jax.ShapeDtypeStruct((NW, 512), jnp.int32),
    mesh=_MESH,
    scratch_types=[
        pltpu.VMEM((BLK,), jnp.int32),
        pltpu.VMEM((BLK,), jnp.int32),
        pltpu.VMEM((BLK,), jnp.int32),
        pltpu.VMEM((BLK,), jnp.int32),
        pltpu.VMEM((512,), jnp.int32),
    ],
)
def _sc_stats(xs, ys, zs, bs, part_out, xbuf, ybuf, zbuf, bbuf, acc):
    wid = _wid()
    zero = jnp.zeros((16,), jnp.int32)
    for q in range(32):
        acc[pl.ds(q * 16, 16)] = zero

    def blk_body(t, carry):
        base = (wid + t * NW) * BLK
        pltpu.sync_copy(xs.at[pl.ds(base, BLK)], xbuf)
        pltpu.sync_copy(ys.at[pl.ds(base, BLK)], ybuf)
        pltpu.sync_copy(zs.at[pl.ds(base, BLK)], zbuf)
        pltpu.sync_copy(bs.at[pl.ds(base, BLK)], bbuf)
        b0 = bbuf[pl.ds(0, 16)][0]
        b1 = bbuf[pl.ds(BLK - 16, 16)][15]

        def fast(_):
            def vb(j, c):
                o = j * 16
                return (c[0] + xbuf[pl.ds(o, 16)],
                        c[1] + ybuf[pl.ds(o, 16)],
                        c[2] + zbuf[pl.ds(o, 16)])

            sx, sy, sz = lax.fori_loop(0, VPB, vb, (zero, zero, zero))
            plsc.addupdate(acc.at[pl.ds(b0 * 16, 16)], sx)
            plsc.addupdate(acc.at[pl.ds(128 + b0 * 16, 16)], sy)
            plsc.addupdate(acc.at[pl.ds(256 + b0 * 16, 16)], sz)
            plsc.addupdate(acc.at[pl.ds(384 + b0 * 16, 16)],
                           jnp.full((16,), VPB, jnp.int32))
            return 0

        def slow(_):
            one = jnp.ones((16,), jnp.int32)

            def vb(j, c):
                o = j * 16
                xv = xbuf[pl.ds(o, 16)]
                yv = ybuf[pl.ds(o, 16)]
                zv = zbuf[pl.ds(o, 16)]
                bv = bbuf[pl.ds(o, 16)]
                for b in range(B):
                    m = bv == b
                    plsc.addupdate(acc.at[pl.ds(b * 16, 16)],
                                   jnp.where(m, xv, zero))
                    plsc.addupdate(acc.at[pl.ds(128 + b * 16, 16)],
                                   jnp.where(m, yv, zero))
                    plsc.addupdate(acc.at[pl.ds(256 + b * 16, 16)],
                                   jnp.where(m, zv, zero))
                    plsc.addupdate(acc.at[pl.ds(384 + b * 16, 16)],
                                   jnp.where(m, one, zero))
                return c

            lax.fori_loop(0, VPB, vb, 0)
            return 0

        lax.cond(b0 == b1, fast, slow, 0)
        return carry

    lax.fori_loop(0, _nblk(wid), blk_body, 0)
    pltpu.sync_copy(acc, part_out.at[wid])


@functools.partial(
    pl.kernel,
    out_type=jax.ShapeDtypeStruct((N,), jnp.int32),
    mesh=_MESH,
    scratch_types=[
        pltpu.VMEM((BLK,), jnp.int32),
        pltpu.VMEM((BLK,), jnp.int32),
        pltpu.VMEM((BLK,), jnp.int32),
        pltpu.VMEM((BLK,), jnp.int32),
        pltpu.VMEM((BLK,), jnp.int32),
        pltpu.VMEM((NW, 512), jnp.int32),
        pltpu.VMEM((384,), jnp.float32),
    ],
)
def _sc_binning(xs, ys, zs, bs, part, seg_out,
                xbuf, ybuf, zbuf, bbuf, segbuf, pbuf, meanbuf):
    wid = _wid()
    pltpu.sync_copy(part, pbuf)

    tot = []
    for q in range(32):
        a = pbuf[0, pl.ds(q * 16, 16)]
        for w in range(1, NW):
            a = a + pbuf[w, pl.ds(q * 16, 16)]
        s = a[0]
        for l in range(1, 16):
            s = s + a[l]
        tot.append(s)
    for b in range(B):
        cnf = jnp.maximum(
            jnp.full((16,), tot[24 + b], jnp.int32).astype(jnp.float32), 1.0)
        meanbuf[pl.ds(b * 16, 16)] = (
            jnp.full((16,), tot[b], jnp.int32).astype(jnp.float32) / cnf)
        meanbuf[pl.ds(128 + b * 16, 16)] = (
            jnp.full((16,), tot[8 + b], jnp.int32).astype(jnp.float32) / cnf)
        meanbuf[pl.ds(256 + b * 16, 16)] = (
            jnp.full((16,), tot[16 + b], jnp.int32).astype(jnp.float32) / cnf)

    def sectors(d2):
        s = jnp.zeros((16,), jnp.int32)
        for k in range(NS):
            s = s + jnp.where(d2 >= _THR[k], 1, 0)
        return s

    def blk_body(t, carry):
        base = (wid + t * NW) * BLK
        pltpu.sync_copy(xs.at[pl.ds(base, BLK)], xbuf)
        pltpu.sync_copy(ys.at[pl.ds(base, BLK)], ybuf)
        pltpu.sync_copy(zs.at[pl.ds(base, BLK)], zbuf)
        pltpu.sync_copy(bs.at[pl.ds(base, BLK)], bbuf)
        b0 = bbuf[pl.ds(0, 16)][0]
        b1 = bbuf[pl.ds(BLK - 16, 16)][15]

        def fast(_):
            mx = meanbuf[pl.ds(b0 * 16, 16)]
            my = meanbuf[pl.ds(128 + b0 * 16, 16)]
            mz = meanbuf[pl.ds(256 + b0 * 16, 16)]
            segbase = jnp.full((16,), b0 * NS, jnp.int32)

            def vb(j, c):
                o = j * 16
                dx = xbuf[pl.ds(o, 16)].astype(jnp.float32) - mx
                dy = ybuf[pl.ds(o, 16)].astype(jnp.float32) - my
                dz = zbuf[pl.ds(o, 16)].astype(jnp.float32) - mz
                s = sectors(dx * dx + dy * dy + dz * dz)
                segbuf[pl.ds(o, 16)] = jnp.where(s < NS, segbase + s, NSEG)
                return c

            lax.fori_loop(0, VPB, vb, 0)
            return 0

        def slow(_):
            def vb(j, c):
                o = j * 16
                bv = bbuf[pl.ds(o, 16)]
                mx = meanbuf[pl.ds(0, 16)]
                my = meanbuf[pl.ds(128, 16)]
                mz = meanbuf[pl.ds(256, 16)]
                for b in range(1, B):
                    m = bv == b
                    mx = jnp.where(m, meanbuf[pl.ds(b * 16, 16)], mx)
                    my = jnp.where(m, meanbuf[pl.ds(128 + b * 16, 16)], my)
                    mz = jnp.where(m, meanbuf[pl.ds(256 + b * 16, 16)], mz)
                dx = xbuf[pl.ds(o, 16)].astype(jnp.float32) - mx
                dy = ybuf[pl.ds(o, 16)].astype(jnp.float32) - my
                dz = zbuf[pl.ds(o, 16)].astype(jnp.float32) - mz
                s = sectors(dx * dx + dy * dy + dz * dz)
                segbuf[pl.ds(o, 16)] = jnp.where(s < NS, bv * NS + s, NSEG)
                return c

            lax.fori_loop(0, VPB, vb, 0)
            return 0

        lax.cond(b0 == b1, fast, slow, 0)
        pltpu.sync_copy(segbuf, seg_out.at[pl.ds(base, BLK)])
        return carry

    lax.fori_loop(0, _nblk(wid), blk_body, 0)


def _feat_kernel(seg_ref, feat_ref, out_ref, facc_ref, cacc_ref):
    i = pl.program_id(0)

    @pl.when(i == 0)
    def _():
        facc_ref[...] = jnp.zeros_like(facc_ref)
        cacc_ref[...] = jnp.zeros_like(cacc_ref)

    seg = seg_ref[...]
    ohs = (jax.lax.broadcasted_iota(jnp.int32, (CHUNK, NSEG), 1) == seg
           ).astype(jnp.bfloat16)
    featb = feat_ref[...].astype(jnp.bfloat16)
    facc_ref[...] += jax.lax.dot_general(
        ohs, featb, (((0,), (0,)), ((), ())),
        preferred_element_type=jnp.float32)
    cacc_ref[...] += jax.lax.dot_general(
        ohs, jnp.ones((CHUNK, 8), jnp.bfloat16), (((0,), (0,)), ((), ())),
        preferred_element_type=jnp.float32)

    @pl.when(i == NCHUNK - 1)
    def _():
        cnt = cacc_ref[:, 0:1]
        out_ref[...] = jnp.where(cnt > 0, facc_ref[...] / jnp.maximum(cnt, 1.0),
                                 0.0)


def kernel(features, coords, batch_indices):
    bi = batch_indices.astype(jnp.int32)
    xs = coords[:, 0]
    ys = coords[:, 1]
    zs = coords[:, 2]
    part = _sc_stats(xs, ys, zs, bi)
    seg = _sc_binning(xs, ys, zs, bi, part)
    out = pl.pallas_call(
        _feat_kernel,
        grid=(NCHUNK,),
        in_specs=[
            pl.BlockSpec((CHUNK, 1), lambda i: (i, 0)),
            pl.BlockSpec((CHUNK, D), lambda i: (i, 0)),
        ],
        out_specs=pl.BlockSpec((NSEG, D), lambda i: (0, 0)),
        out_shape=jax.ShapeDtypeStruct((NSEG, D), jnp.float32),
        scratch_shapes=[
            pltpu.VMEM((NSEG, D), jnp.float32),
            pltpu.VMEM((NSEG, 8), jnp.float32),
        ],
    )(seg.reshape(N, 1), features)
    return out.reshape(B, NS * D)


# speedup vs baseline: 14.7890x; 1.6323x over previous
import functools

import jax
import jax.numpy as jnp
import numpy as np
from jax import lax
from jax.experimental import pallas as pl
from jax.experimental.pallas import tpu as pltpu
from jax.experimental.pallas import tpu_sc as plsc

N = 1000000
B = 8
D = 64
NS = 16
NSEG = B * NS

NW = 32
BLK = 2000
NBLK = N // BLK
VPB = BLK // 16
NBLK_BASE = NBLK // NW
NBLK_EXTRA = NBLK - NBLK_BASE * NW

CHUNK = 4000
NCHUNK = N // CHUNK


def _exact_d2_thresholds():
    v05 = np.float32(0.05)
    thr = []
    for k in range(1, 17):
        rk = np.float32(4.0 + 3.25 * k)

        def pred(v):
            return np.float32(np.sqrt(np.float32(v)) * v05) >= rk

        v = np.float32((80.0 + 65.0 * k) ** 2)
        if pred(v):
            while pred(np.nextafter(v, np.float32(0.0))):
                v = np.nextafter(v, np.float32(0.0))
        else:
            while not pred(v):
                v = np.nextafter(v, np.float32(np.inf))
        thr.append(float(v))
    return thr


_THR = _exact_d2_thresholds()

_MESH = plsc.VectorSubcoreMesh(core_axis_name="c", subcore_axis_name="s")


def _wid():
    return lax.axis_index("s") * 2 + lax.axis_index("c")


def _nblk(wid):
    return NBLK_BASE + jnp.where(wid < NBLK_EXTRA, 1, 0)


@functools.partial(
    pl.kernel,
    out_type=jax.ShapeDtypeStruct((NW, 512), jnp.int32),
    mesh=_MESH,
    scratch_types=[
        pltpu.VMEM((BLK,), jnp.int32),
        pltpu.VMEM((BLK,), jnp.int32),
        pltpu.VMEM((BLK,), jnp.int32),
        pltpu.VMEM((BLK,), jnp.int32),
        pltpu.VMEM((512,), jnp.int32),
    ],
)
def _sc_stats(xs, ys, zs, bs, part_out, xbuf, ybuf, zbuf, bbuf, acc):
    wid = _wid()
    zero = jnp.zeros((16,), jnp.int32)
    for q in range(32):
        acc[pl.ds(q * 16, 16)] = zero

    def blk_body(t, carry):
        base = (wid + t * NW) * BLK
        pltpu.sync_copy(xs.at[pl.ds(base, BLK)], xbuf)
        pltpu.sync_copy(ys.at[pl.ds(base, BLK)], ybuf)
        pltpu.sync_copy(zs.at[pl.ds(base, BLK)], zbuf)
        pltpu.sync_copy(bs.at[pl.ds(base, BLK)], bbuf)
        b0 = bbuf[pl.ds(0, 16)][0]
        b1 = bbuf[pl.ds(BLK - 16, 16)][15]

        def fast(_):
            def vb(j, c):
                o = j * 16
                return (c[0] + xbuf[pl.ds(o, 16)],
                        c[1] + ybuf[pl.ds(o, 16)],
                        c[2] + zbuf[pl.ds(o, 16)])

            sx, sy, sz = lax.fori_loop(0, VPB, vb, (zero, zero, zero))
            plsc.addupdate(acc.at[pl.ds(b0 * 16, 16)], sx)
            plsc.addupdate(acc.at[pl.ds(128 + b0 * 16, 16)], sy)
            plsc.addupdate(acc.at[pl.ds(256 + b0 * 16, 16)], sz)
            plsc.addupdate(acc.at[pl.ds(384 + b0 * 16, 16)],
                           jnp.full((16,), VPB, jnp.int32))
            return 0

        def slow(_):
            one = jnp.ones((16,), jnp.int32)

            def vb(j, c):
                o = j * 16
                xv = xbuf[pl.ds(o, 16)]
                yv = ybuf[pl.ds(o, 16)]
                zv = zbuf[pl.ds(o, 16)]
                bv = bbuf[pl.ds(o, 16)]
                for b in range(B):
                    m = bv == b
                    plsc.addupdate(acc.at[pl.ds(b * 16, 16)],
                                   jnp.where(m, xv, zero))
                    plsc.addupdate(acc.at[pl.ds(128 + b * 16, 16)],
                                   jnp.where(m, yv, zero))
                    plsc.addupdate(acc.at[pl.ds(256 + b * 16, 16)],
                                   jnp.where(m, zv, zero))
                    plsc.addupdate(acc.at[pl.ds(384 + b * 16, 16)],
                                   jnp.where(m, one, zero))
                return c

            lax.fori_loop(0, VPB, vb, 0)
            return 0

        lax.cond(b0 == b1, fast, slow, 0)
        return carry

    lax.fori_loop(0, _nblk(wid), blk_body, 0)
    pltpu.sync_copy(acc, part_out.at[wid])


@functools.partial(
    pl.kernel,
    out_type=jax.ShapeDtypeStruct((N,), jnp.int32),
    mesh=_MESH,
    scratch_types=[
        pltpu.VMEM((BLK,), jnp.int32),
        pltpu.VMEM((BLK,), jnp.int32),
        pltpu.VMEM((BLK,), jnp.int32),
        pltpu.VMEM((BLK,), jnp.int32),
        pltpu.VMEM((BLK,), jnp.int32),
        pltpu.VMEM((NW, 512), jnp.int32),
        pltpu.VMEM((384,), jnp.float32),
    ],
)
def _sc_binning(xs, ys, zs, bs, part, seg_out,
                xbuf, ybuf, zbuf, bbuf, segbuf, pbuf, meanbuf):
    wid = _wid()
    pltpu.sync_copy(part, pbuf)

    tot = []
    for q in range(32):
        a = pbuf[0, pl.ds(q * 16, 16)]
        for w in range(1, NW):
            a = a + pbuf[w, pl.ds(q * 16, 16)]
        s = a[0]
        for l in range(1, 16):
            s = s + a[l]
        tot.append(s)
    for b in range(B):
        cnf = jnp.maximum(
            jnp.full((16,), tot[24 + b], jnp.int32).astype(jnp.float32), 1.0)
        meanbuf[pl.ds(b * 16, 16)] = (
            jnp.full((16,), tot[b], jnp.int32).astype(jnp.float32) / cnf)
        meanbuf[pl.ds(128 + b * 16, 16)] = (
            jnp.full((16,), tot[8 + b], jnp.int32).astype(jnp.float32) / cnf)
        meanbuf[pl.ds(256 + b * 16, 16)] = (
            jnp.full((16,), tot[16 + b], jnp.int32).astype(jnp.float32) / cnf)

    def sectors(d2):
        s = jnp.zeros((16,), jnp.int32)
        for k in range(NS):
            s = s + jnp.where(d2 >= _THR[k], 1, 0)
        return s

    def blk_body(t, carry):
        base = (wid + t * NW) * BLK
        pltpu.sync_copy(xs.at[pl.ds(base, BLK)], xbuf)
        pltpu.sync_copy(ys.at[pl.ds(base, BLK)], ybuf)
        pltpu.sync_copy(zs.at[pl.ds(base, BLK)], zbuf)
        pltpu.sync_copy(bs.at[pl.ds(base, BLK)], bbuf)
        b0 = bbuf[pl.ds(0, 16)][0]
        b1 = bbuf[pl.ds(BLK - 16, 16)][15]

        def fast(_):
            mx = meanbuf[pl.ds(b0 * 16, 16)]
            my = meanbuf[pl.ds(128 + b0 * 16, 16)]
            mz = meanbuf[pl.ds(256 + b0 * 16, 16)]
            segbase = jnp.full((16,), b0 * NS, jnp.int32)

            def vb(j, c):
                o = j * 16
                dx = xbuf[pl.ds(o, 16)].astype(jnp.float32) - mx
                dy = ybuf[pl.ds(o, 16)].astype(jnp.float32) - my
                dz = zbuf[pl.ds(o, 16)].astype(jnp.float32) - mz
                s = sectors(dx * dx + dy * dy + dz * dz)
                segbuf[pl.ds(o, 16)] = jnp.where(s < NS, segbase + s, NSEG)
                return c

            lax.fori_loop(0, VPB, vb, 0)
            return 0

        def slow(_):
            def vb(j, c):
                o = j * 16
                bv = bbuf[pl.ds(o, 16)]
                mx = meanbuf[pl.ds(0, 16)]
                my = meanbuf[pl.ds(128, 16)]
                mz = meanbuf[pl.ds(256, 16)]
                for b in range(1, B):
                    m = bv == b
                    mx = jnp.where(m, meanbuf[pl.ds(b * 16, 16)], mx)
                    my = jnp.where(m, meanbuf[pl.ds(128 + b * 16, 16)], my)
                    mz = jnp.where(m, meanbuf[pl.ds(256 + b * 16, 16)], mz)
                dx = xbuf[pl.ds(o, 16)].astype(jnp.float32) - mx
                dy = ybuf[pl.ds(o, 16)].astype(jnp.float32) - my
                dz = zbuf[pl.ds(o, 16)].astype(jnp.float32) - mz
                s = sectors(dx * dx + dy * dy + dz * dz)
                segbuf[pl.ds(o, 16)] = jnp.where(s < NS, bv * NS + s, NSEG)
                return c

            lax.fori_loop(0, VPB, vb, 0)
            return 0

        lax.cond(b0 == b1, fast, slow, 0)
        pltpu.sync_copy(segbuf, seg_out.at[pl.ds(base, BLK)])
        return carry

    lax.fori_loop(0, _nblk(wid), blk_body, 0)


def _feat_kernel(seg_ref, feat_ref, out_ref, facc_ref, cacc_ref):
    i = pl.program_id(0)

    @pl.when(i == 0)
    def _():
        facc_ref[...] = jnp.zeros_like(facc_ref)
        cacc_ref[...] = jnp.zeros_like(cacc_ref)

    seg = seg_ref[0]
    ohs = (jax.lax.broadcasted_iota(jnp.int32, (NSEG, CHUNK), 0) == seg
           ).astype(jnp.bfloat16)
    featb = feat_ref[...].astype(jnp.bfloat16)
    facc_ref[...] += jax.lax.dot_general(
        ohs, featb, (((1,), (0,)), ((), ())),
        preferred_element_type=jnp.float32)
    cacc_ref[...] += jax.lax.dot_general(
        ohs, jnp.ones((CHUNK, 8), jnp.bfloat16), (((1,), (0,)), ((), ())),
        preferred_element_type=jnp.float32)

    @pl.when(i == NCHUNK - 1)
    def _():
        cnt = cacc_ref[:, 0:1]
        out_ref[...] = jnp.where(cnt > 0, facc_ref[...] / jnp.maximum(cnt, 1.0),
                                 0.0)


def kernel(features, coords, batch_indices):
    bi = batch_indices.astype(jnp.int32)
    xs = coords[:, 0]
    ys = coords[:, 1]
    zs = coords[:, 2]
    part = _sc_stats(xs, ys, zs, bi)
    seg = _sc_binning(xs, ys, zs, bi, part)
    out = pl.pallas_call(
        _feat_kernel,
        grid=(NCHUNK,),
        in_specs=[
            pl.BlockSpec((1, 1, CHUNK), lambda i: (i, 0, 0)),
            pl.BlockSpec((CHUNK, D), lambda i: (i, 0)),
        ],
        out_specs=pl.BlockSpec((NSEG, D), lambda i: (0, 0)),
        out_shape=jax.ShapeDtypeStruct((NSEG, D), jnp.float32),
        scratch_shapes=[
            pltpu.VMEM((NSEG, D), jnp.float32),
            pltpu.VMEM((NSEG, 8), jnp.float32),
        ],
    )(seg.reshape(NCHUNK, 1, CHUNK), features)
    return out.reshape(B, NS * D)


# speedup vs baseline: 16.7272x; 1.1311x over previous
import functools

import jax
import jax.numpy as jnp
import numpy as np
from jax import lax
from jax.experimental import pallas as pl
from jax.experimental.pallas import tpu as pltpu
from jax.experimental.pallas import tpu_sc as plsc

N = 1000000
B = 8
D = 64
NS = 16
NSEG = B * NS

NW = 32
BLK = 2000
NBLK = N // BLK
VPB = BLK // 16
NBLK_BASE = NBLK // NW
NBLK_EXTRA = NBLK - NBLK_BASE * NW

CHUNK = 10000
NCHUNK = N // CHUNK


def _exact_d2_thresholds():
    v05 = np.float32(0.05)
    thr = []
    for k in range(1, 17):
        rk = np.float32(4.0 + 3.25 * k)

        def pred(v):
            return np.float32(np.sqrt(np.float32(v)) * v05) >= rk

        v = np.float32((80.0 + 65.0 * k) ** 2)
        if pred(v):
            while pred(np.nextafter(v, np.float32(0.0))):
                v = np.nextafter(v, np.float32(0.0))
        else:
            while not pred(v):
                v = np.nextafter(v, np.float32(np.inf))
        thr.append(float(v))
    return thr


_THR = _exact_d2_thresholds()

_MESH = plsc.VectorSubcoreMesh(core_axis_name="c", subcore_axis_name="s")


def _wid():
    return lax.axis_index("s") * 2 + lax.axis_index("c")


def _nblk(wid):
    return NBLK_BASE + jnp.where(wid < NBLK_EXTRA, 1, 0)


@functools.partial(
    pl.kernel,
    out_type=jax.ShapeDtypeStruct((NW, 512), jnp.int32),
    mesh=_MESH,
    scratch_types=[
        pltpu.VMEM((BLK,), jnp.int32),
        pltpu.VMEM((BLK,), jnp.int32),
        pltpu.VMEM((BLK,), jnp.int32),
        pltpu.VMEM((BLK,), jnp.int32),
        pltpu.VMEM((512,), jnp.int32),
    ],
)
def _sc_stats(xs, ys, zs, bs, part_out, xbuf, ybuf, zbuf, bbuf, acc):
    wid = _wid()
    zero = jnp.zeros((16,), jnp.int32)
    for q in range(32):
        acc[pl.ds(q * 16, 16)] = zero

    def blk_body(t, carry):
        base = (wid + t * NW) * BLK
        pltpu.sync_copy(xs.at[pl.ds(base, BLK)], xbuf)
        pltpu.sync_copy(ys.at[pl.ds(base, BLK)], ybuf)
        pltpu.sync_copy(zs.at[pl.ds(base, BLK)], zbuf)
        pltpu.sync_copy(bs.at[pl.ds(base, BLK)], bbuf)
        b0 = bbuf[pl.ds(0, 16)][0]
        b1 = bbuf[pl.ds(BLK - 16, 16)][15]

        def fast(_):
            def vb(j, c):
                o = j * 16
                return (c[0] + xbuf[pl.ds(o, 16)],
                        c[1] + ybuf[pl.ds(o, 16)],
                        c[2] + zbuf[pl.ds(o, 16)])

            sx, sy, sz = lax.fori_loop(0, VPB, vb, (zero, zero, zero))
            plsc.addupdate(acc.at[pl.ds(b0 * 16, 16)], sx)
            plsc.addupdate(acc.at[pl.ds(128 + b0 * 16, 16)], sy)
            plsc.addupdate(acc.at[pl.ds(256 + b0 * 16, 16)], sz)
            plsc.addupdate(acc.at[pl.ds(384 + b0 * 16, 16)],
                           jnp.full((16,), VPB, jnp.int32))
            return 0

        def slow(_):
            one = jnp.ones((16,), jnp.int32)

            def vb(j, c):
                o = j * 16
                xv = xbuf[pl.ds(o, 16)]
                yv = ybuf[pl.ds(o, 16)]
                zv = zbuf[pl.ds(o, 16)]
                bv = bbuf[pl.ds(o, 16)]
                for b in range(B):
                    m = bv == b
                    plsc.addupdate(acc.at[pl.ds(b * 16, 16)],
                                   jnp.where(m, xv, zero))
                    plsc.addupdate(acc.at[pl.ds(128 + b * 16, 16)],
                                   jnp.where(m, yv, zero))
                    plsc.addupdate(acc.at[pl.ds(256 + b * 16, 16)],
                                   jnp.where(m, zv, zero))
                    plsc.addupdate(acc.at[pl.ds(384 + b * 16, 16)],
                                   jnp.where(m, one, zero))
                return c

            lax.fori_loop(0, VPB, vb, 0)
            return 0

        lax.cond(b0 == b1, fast, slow, 0)
        return carry

    lax.fori_loop(0, _nblk(wid), blk_body, 0)
    pltpu.sync_copy(acc, part_out.at[wid])


@functools.partial(
    pl.kernel,
    out_type=jax.ShapeDtypeStruct((N,), jnp.int32),
    mesh=_MESH,
    scratch_types=[
        pltpu.VMEM((BLK,), jnp.int32),
        pltpu.VMEM((BLK,), jnp.int32),
        pltpu.VMEM((BLK,), jnp.int32),
        pltpu.VMEM((BLK,), jnp.int32),
        pltpu.VMEM((BLK,), jnp.int32),
        pltpu.VMEM((NW, 512), jnp.int32),
        pltpu.VMEM((384,), jnp.float32),
    ],
)
def _sc_binning(xs, ys, zs, bs, part, seg_out,
                xbuf, ybuf, zbuf, bbuf, segbuf, pbuf, meanbuf):
    wid = _wid()
    pltpu.sync_copy(part, pbuf)

    tot = []
    for q in range(32):
        a = pbuf[0, pl.ds(q * 16, 16)]
        for w in range(1, NW):
            a = a + pbuf[w, pl.ds(q * 16, 16)]
        s = a[0]
        for l in range(1, 16):
            s = s + a[l]
        tot.append(s)
    for b in range(B):
        cnf = jnp.maximum(
            jnp.full((16,), tot[24 + b], jnp.int32).astype(jnp.float32), 1.0)
        meanbuf[pl.ds(b * 16, 16)] = (
            jnp.full((16,), tot[b], jnp.int32).astype(jnp.float32) / cnf)
        meanbuf[pl.ds(128 + b * 16, 16)] = (
            jnp.full((16,), tot[8 + b], jnp.int32).astype(jnp.float32) / cnf)
        meanbuf[pl.ds(256 + b * 16, 16)] = (
            jnp.full((16,), tot[16 + b], jnp.int32).astype(jnp.float32) / cnf)

    def sectors(d2):
        s = jnp.zeros((16,), jnp.int32)
        for k in range(NS):
            s = s + jnp.where(d2 >= _THR[k], 1, 0)
        return s

    def blk_body(t, carry):
        base = (wid + t * NW) * BLK
        pltpu.sync_copy(xs.at[pl.ds(base, BLK)], xbuf)
        pltpu.sync_copy(ys.at[pl.ds(base, BLK)], ybuf)
        pltpu.sync_copy(zs.at[pl.ds(base, BLK)], zbuf)
        pltpu.sync_copy(bs.at[pl.ds(base, BLK)], bbuf)
        b0 = bbuf[pl.ds(0, 16)][0]
        b1 = bbuf[pl.ds(BLK - 16, 16)][15]

        def fast(_):
            mx = meanbuf[pl.ds(b0 * 16, 16)]
            my = meanbuf[pl.ds(128 + b0 * 16, 16)]
            mz = meanbuf[pl.ds(256 + b0 * 16, 16)]
            segbase = jnp.full((16,), b0 * NS, jnp.int32)

            def vb(j, c):
                o = j * 16
                dx = xbuf[pl.ds(o, 16)].astype(jnp.float32) - mx
                dy = ybuf[pl.ds(o, 16)].astype(jnp.float32) - my
                dz = zbuf[pl.ds(o, 16)].astype(jnp.float32) - mz
                s = sectors(dx * dx + dy * dy + dz * dz)
                segbuf[pl.ds(o, 16)] = jnp.where(s < NS, segbase + s, NSEG)
                return c

            lax.fori_loop(0, VPB, vb, 0)
            return 0

        def slow(_):
            def vb(j, c):
                o = j * 16
                bv = bbuf[pl.ds(o, 16)]
                mx = meanbuf[pl.ds(0, 16)]
                my = meanbuf[pl.ds(128, 16)]
                mz = meanbuf[pl.ds(256, 16)]
                for b in range(1, B):
                    m = bv == b
                    mx = jnp.where(m, meanbuf[pl.ds(b * 16, 16)], mx)
                    my = jnp.where(m, meanbuf[pl.ds(128 + b * 16, 16)], my)
                    mz = jnp.where(m, meanbuf[pl.ds(256 + b * 16, 16)], mz)
                dx = xbuf[pl.ds(o, 16)].astype(jnp.float32) - mx
                dy = ybuf[pl.ds(o, 16)].astype(jnp.float32) - my
                dz = zbuf[pl.ds(o, 16)].astype(jnp.float32) - mz
                s = sectors(dx * dx + dy * dy + dz * dz)
                segbuf[pl.ds(o, 16)] = jnp.where(s < NS, bv * NS + s, NSEG)
                return c

            lax.fori_loop(0, VPB, vb, 0)
            return 0

        lax.cond(b0 == b1, fast, slow, 0)
        pltpu.sync_copy(segbuf, seg_out.at[pl.ds(base, BLK)])
        return carry

    lax.fori_loop(0, _nblk(wid), blk_body, 0)


def _feat_kernel(seg_ref, feat_ref, out_ref, facc_ref, cacc_ref):
    i = pl.program_id(0)

    @pl.when(i == 0)
    def _():
        facc_ref[...] = jnp.zeros_like(facc_ref)
        cacc_ref[...] = jnp.zeros_like(cacc_ref)

    seg = seg_ref[0]
    ohs = (jax.lax.broadcasted_iota(jnp.int32, (NSEG, CHUNK), 0) == seg
           ).astype(jnp.bfloat16)
    featb = feat_ref[...].astype(jnp.bfloat16)
    facc_ref[...] += jax.lax.dot_general(
        ohs, featb, (((1,), (0,)), ((), ())),
        preferred_element_type=jnp.float32)
    cacc_ref[...] += jax.lax.dot_general(
        ohs, jnp.ones((CHUNK, 8), jnp.bfloat16), (((1,), (0,)), ((), ())),
        preferred_element_type=jnp.float32)

    @pl.when(i == NCHUNK - 1)
    def _():
        cnt = cacc_ref[:, 0:1]
        out_ref[...] = jnp.where(cnt > 0, facc_ref[...] / jnp.maximum(cnt, 1.0),
                                 0.0)


def kernel(features, coords, batch_indices):
    bi = batch_indices.astype(jnp.int32)
    xs = coords[:, 0]
    ys = coords[:, 1]
    zs = coords[:, 2]
    part = _sc_stats(xs, ys, zs, bi)
    seg = _sc_binning(xs, ys, zs, bi, part)
    out = pl.pallas_call(
        _feat_kernel,
        grid=(NCHUNK,),
        in_specs=[
            pl.BlockSpec((1, 1, CHUNK), lambda i: (i, 0, 0)),
            pl.BlockSpec((CHUNK, D), lambda i: (i, 0)),
        ],
        out_specs=pl.BlockSpec((NSEG, D), lambda i: (0, 0)),
        out_shape=jax.ShapeDtypeStruct((NSEG, D), jnp.float32),
        scratch_shapes=[
            pltpu.VMEM((NSEG, D), jnp.float32),
            pltpu.VMEM((NSEG, 8), jnp.float32),
        ],
    )(seg.reshape(NCHUNK, 1, CHUNK), features)
    return out.reshape(B, NS * D)


# speedup vs baseline: 17.5416x; 1.0487x over previous
import functools

import jax
import jax.numpy as jnp
import numpy as np
from jax import lax
from jax.experimental import pallas as pl
from jax.experimental.pallas import tpu as pltpu
from jax.experimental.pallas import tpu_sc as plsc

N = 1000000
B = 8
D = 64
NS = 16
NSEG = B * NS

NW = 32
BLK = 2000
NBLK = N // BLK
VPB = BLK // 16
NBLK_BASE = NBLK // NW
NBLK_EXTRA = NBLK - NBLK_BASE * NW

CHUNK = 20000
NCHUNK = N // CHUNK


def _exact_d2_thresholds():
    v05 = np.float32(0.05)
    thr = []
    for k in range(1, 17):
        rk = np.float32(4.0 + 3.25 * k)

        def pred(v):
            return np.float32(np.sqrt(np.float32(v)) * v05) >= rk

        v = np.float32((80.0 + 65.0 * k) ** 2)
        if pred(v):
            while pred(np.nextafter(v, np.float32(0.0))):
                v = np.nextafter(v, np.float32(0.0))
        else:
            while not pred(v):
                v = np.nextafter(v, np.float32(np.inf))
        thr.append(float(v))
    return thr


_THR = _exact_d2_thresholds()

_MESH = plsc.VectorSubcoreMesh(core_axis_name="c", subcore_axis_name="s")


def _wid():
    return lax.axis_index("s") * 2 + lax.axis_index("c")


def _nblk(wid):
    return NBLK_BASE + jnp.where(wid < NBLK_EXTRA, 1, 0)


@functools.partial(
    pl.kernel,
    out_type=jax.ShapeDtypeStruct((NW, 512), jnp.int32),
    mesh=_MESH,
    scratch_types=[
        pltpu.VMEM((BLK,), jnp.int32),
        pltpu.VMEM((BLK,), jnp.int32),
        pltpu.VMEM((BLK,), jnp.int32),
        pltpu.VMEM((BLK,), jnp.int32),
        pltpu.VMEM((512,), jnp.int32),
    ],
)
def _sc_stats(xs, ys, zs, bs, part_out, xbuf, ybuf, zbuf, bbuf, acc):
    wid = _wid()
    zero = jnp.zeros((16,), jnp.int32)
    for q in range(32):
        acc[pl.ds(q * 16, 16)] = zero

    def blk_body(t, carry):
        base = (wid + t * NW) * BLK
        pltpu.sync_copy(xs.at[pl.ds(base, BLK)], xbuf)
        pltpu.sync_copy(ys.at[pl.ds(base, BLK)], ybuf)
        pltpu.sync_copy(zs.at[pl.ds(base, BLK)], zbuf)
        pltpu.sync_copy(bs.at[pl.ds(base, BLK)], bbuf)
        b0 = bbuf[pl.ds(0, 16)][0]
        b1 = bbuf[pl.ds(BLK - 16, 16)][15]

        def fast(_):
            def vb(j, c):
                o = j * 16
                return (c[0] + xbuf[pl.ds(o, 16)],
                        c[1] + ybuf[pl.ds(o, 16)],
                        c[2] + zbuf[pl.ds(o, 16)])

            sx, sy, sz = lax.fori_loop(0, VPB, vb, (zero, zero, zero))
            plsc.addupdate(acc.at[pl.ds(b0 * 16, 16)], sx)
            plsc.addupdate(acc.at[pl.ds(128 + b0 * 16, 16)], sy)
            plsc.addupdate(acc.at[pl.ds(256 + b0 * 16, 16)], sz)
            plsc.addupdate(acc.at[pl.ds(384 + b0 * 16, 16)],
                           jnp.full((16,), VPB, jnp.int32))
            return 0

        def slow(_):
            one = jnp.ones((16,), jnp.int32)

            def vb(j, c):
                o = j * 16
                xv = xbuf[pl.ds(o, 16)]
                yv = ybuf[pl.ds(o, 16)]
                zv = zbuf[pl.ds(o, 16)]
                bv = bbuf[pl.ds(o, 16)]
                for b in range(B):
                    m = bv == b
                    plsc.addupdate(acc.at[pl.ds(b * 16, 16)],
                                   jnp.where(m, xv, zero))
                    plsc.addupdate(acc.at[pl.ds(128 + b * 16, 16)],
                                   jnp.where(m, yv, zero))
                    plsc.addupdate(acc.at[pl.ds(256 + b * 16, 16)],
                                   jnp.where(m, zv, zero))
                    plsc.addupdate(acc.at[pl.ds(384 + b * 16, 16)],
                                   jnp.where(m, one, zero))
                return c

            lax.fori_loop(0, VPB, vb, 0)
            return 0

        lax.cond(b0 == b1, fast, slow, 0)
        return carry

    lax.fori_loop(0, _nblk(wid), blk_body, 0)
    pltpu.sync_copy(acc, part_out.at[wid])


@functools.partial(
    pl.kernel,
    out_type=jax.ShapeDtypeStruct((N,), jnp.int32),
    mesh=_MESH,
    scratch_types=[
        pltpu.VMEM((BLK,), jnp.int32),
        pltpu.VMEM((BLK,), jnp.int32),
        pltpu.VMEM((BLK,), jnp.int32),
        pltpu.VMEM((BLK,), jnp.int32),
        pltpu.VMEM((BLK,), jnp.int32),
        pltpu.VMEM((NW, 512), jnp.int32),
        pltpu.VMEM((384,), jnp.float32),
    ],
)
def _sc_binning(xs, ys, zs, bs, part, seg_out,
                xbuf, ybuf, zbuf, bbuf, segbuf, pbuf, meanbuf):
    wid = _wid()
    pltpu.sync_copy(part, pbuf)

    tot = []
    for q in range(32):
        a = pbuf[0, pl.ds(q * 16, 16)]
        for w in range(1, NW):
            a = a + pbuf[w, pl.ds(q * 16, 16)]
        s = a[0]
        for l in range(1, 16):
            s = s + a[l]
        tot.append(s)
    for b in range(B):
        cnf = jnp.maximum(
            jnp.full((16,), tot[24 + b], jnp.int32).astype(jnp.float32), 1.0)
        meanbuf[pl.ds(b * 16, 16)] = (
            jnp.full((16,), tot[b], jnp.int32).astype(jnp.float32) / cnf)
        meanbuf[pl.ds(128 + b * 16, 16)] = (
            jnp.full((16,), tot[8 + b], jnp.int32).astype(jnp.float32) / cnf)
        meanbuf[pl.ds(256 + b * 16, 16)] = (
            jnp.full((16,), tot[16 + b], jnp.int32).astype(jnp.float32) / cnf)

    def sectors(d2):
        s = jnp.zeros((16,), jnp.int32)
        for k in range(NS):
            s = s + jnp.where(d2 >= _THR[k], 1, 0)
        return s

    def blk_body(t, carry):
        base = (wid + t * NW) * BLK
        pltpu.sync_copy(xs.at[pl.ds(base, BLK)], xbuf)
        pltpu.sync_copy(ys.at[pl.ds(base, BLK)], ybuf)
        pltpu.sync_copy(zs.at[pl.ds(base, BLK)], zbuf)
        pltpu.sync_copy(bs.at[pl.ds(base, BLK)], bbuf)
        b0 = bbuf[pl.ds(0, 16)][0]
        b1 = bbuf[pl.ds(BLK - 16, 16)][15]

        def fast(_):
            mx = meanbuf[pl.ds(b0 * 16, 16)]
            my = meanbuf[pl.ds(128 + b0 * 16, 16)]
            mz = meanbuf[pl.ds(256 + b0 * 16, 16)]
            segbase = jnp.full((16,), b0 * NS, jnp.int32)

            def vb(j, c):
                o = j * 16
                dx = xbuf[pl.ds(o, 16)].astype(jnp.float32) - mx
                dy = ybuf[pl.ds(o, 16)].astype(jnp.float32) - my
                dz = zbuf[pl.ds(o, 16)].astype(jnp.float32) - mz
                s = sectors(dx * dx + dy * dy + dz * dz)
                segbuf[pl.ds(o, 16)] = jnp.where(s < NS, segbase + s, NSEG)
                return c

            lax.fori_loop(0, VPB, vb, 0)
            return 0

        def slow(_):
            def vb(j, c):
                o = j * 16
                bv = bbuf[pl.ds(o, 16)]
                mx = meanbuf[pl.ds(0, 16)]
                my = meanbuf[pl.ds(128, 16)]
                mz = meanbuf[pl.ds(256, 16)]
                for b in range(1, B):
                    m = bv == b
                    mx = jnp.where(m, meanbuf[pl.ds(b * 16, 16)], mx)
                    my = jnp.where(m, meanbuf[pl.ds(128 + b * 16, 16)], my)
                    mz = jnp.where(m, meanbuf[pl.ds(256 + b * 16, 16)], mz)
                dx = xbuf[pl.ds(o, 16)].astype(jnp.float32) - mx
                dy = ybuf[pl.ds(o, 16)].astype(jnp.float32) - my
                dz = zbuf[pl.ds(o, 16)].astype(jnp.float32) - mz
                s = sectors(dx * dx + dy * dy + dz * dz)
                segbuf[pl.ds(o, 16)] = jnp.where(s < NS, bv * NS + s, NSEG)
                return c

            lax.fori_loop(0, VPB, vb, 0)
            return 0

        lax.cond(b0 == b1, fast, slow, 0)
        pltpu.sync_copy(segbuf, seg_out.at[pl.ds(base, BLK)])
        return carry

    lax.fori_loop(0, _nblk(wid), blk_body, 0)


def _feat_kernel(seg_ref, feat_ref, out_ref, facc_ref, cacc_ref):
    i = pl.program_id(0)

    @pl.when(i == 0)
    def _():
        facc_ref[...] = jnp.zeros_like(facc_ref)
        cacc_ref[...] = jnp.zeros_like(cacc_ref)

    seg = seg_ref[0]
    ohs = (jax.lax.broadcasted_iota(jnp.int32, (NSEG, CHUNK), 0) == seg
           ).astype(jnp.bfloat16)
    featb = feat_ref[...].astype(jnp.bfloat16)
    facc_ref[...] += jax.lax.dot_general(
        ohs, featb, (((1,), (0,)), ((), ())),
        preferred_element_type=jnp.float32)
    cacc_ref[...] += jax.lax.dot_general(
        ohs, jnp.ones((CHUNK, 8), jnp.bfloat16), (((1,), (0,)), ((), ())),
        preferred_element_type=jnp.float32)

    @pl.when(i == NCHUNK - 1)
    def _():
        cnt = cacc_ref[:, 0:1]
        out_ref[...] = jnp.where(cnt > 0, facc_ref[...] / jnp.maximum(cnt, 1.0),
                                 0.0)


def kernel(features, coords, batch_indices):
    bi = batch_indices.astype(jnp.int32)
    xs = coords[:, 0]
    ys = coords[:, 1]
    zs = coords[:, 2]
    part = _sc_stats(xs, ys, zs, bi)
    seg = _sc_binning(xs, ys, zs, bi, part)
    out = pl.pallas_call(
        _feat_kernel,
        grid=(NCHUNK,),
        in_specs=[
            pl.BlockSpec((1, 1, CHUNK), lambda i: (i, 0, 0)),
            pl.BlockSpec((CHUNK, D), lambda i: (i, 0)),
        ],
        out_specs=pl.BlockSpec((NSEG, D), lambda i: (0, 0)),
        out_shape=jax.ShapeDtypeStruct((NSEG, D), jnp.float32),
        scratch_shapes=[
            pltpu.VMEM((NSEG, D), jnp.float32),
            pltpu.VMEM((NSEG, 8), jnp.float32),
        ],
    )(seg.reshape(NCHUNK, 1, CHUNK), features)
    return out.reshape(B, NS * D)
